# baseline (speedup 1.0000x reference)
import functools

import jax
import jax.numpy as jnp
from jax import lax
from jax.experimental import pallas as pl
from jax.experimental.pallas import tpu as pltpu
from jax.experimental.pallas import tpu_sc as plsc

_NC = 2
_NS = 16
_LANES = 16
_NW = _NC * _NS
_VPAD = 128


def _sc_counts(words, B, W):
    bags_per_w = B // _NW
    n_chunks = 2
    chunk_b = bags_per_w // n_chunks
    groups_per_chunk = chunk_b // _LANES

    mesh = plsc.VectorSubcoreMesh(
        core_axis_name="c", subcore_axis_name="s",
        num_cores=_NC, num_subcores=_NS)

    @functools.partial(
        pl.kernel,
        mesh=mesh,
        out_type=jax.ShapeDtypeStruct((B, _VPAD), jnp.float32),
        scratch_types=[
            pltpu.VMEM((chunk_b, W), jnp.int32),
            pltpu.VMEM((chunk_b, W), jnp.int32),
            pltpu.VMEM((bags_per_w, _VPAD), jnp.float32),
            pltpu.SemaphoreType.DMA,
            pltpu.SemaphoreType.DMA,
        ],
        compiler_params=pltpu.CompilerParams(
            needs_layout_passes=False, use_tc_tiling_on_sc=False),
    )
    def counts_kernel(idx_hbm, counts_hbm, idx_v0, idx_v1, cnt_v, sem0, sem1):
        wid = lax.axis_index("s") * _NC + lax.axis_index("c")
        base = wid * bags_per_w
        bufs = (idx_v0, idx_v1)
        sems = (sem0, sem1)

        def start_load(c):
            return pltpu.async_copy(
                idx_hbm.at[pl.ds(base + c * chunk_b, chunk_b), :],
                bufs[c % 2], sems[c % 2])

        cps = [start_load(0), start_load(1)]

        rows16 = lax.iota(jnp.int32, 16)
        ones16 = jnp.ones((16,), jnp.float32)
        zeros16 = jnp.zeros((16,), jnp.float32)

        @plsc.parallel_loop(0, bags_per_w, unroll=2)
        def _zero(r):
            for c in range(_VPAD // 16):
                cnt_v[r, pl.ds(c * 16, 16)] = zeros16

        for chunk in range(n_chunks):
            cps[chunk].wait()
            idx_v = bufs[chunk % 2]

            def group_body(g, carry, idx_v=idx_v, chunk=chunk):
                grows = g * _LANES + rows16
                crows = chunk * chunk_b + grows

                @plsc.parallel_loop(0, W, unroll=4)
                def _accum(w):
                    col = jnp.full((16,), w, jnp.int32)
                    wv = plsc.load_gather(idx_v, [grows, col])
                    for k in range(4):
                        bk = (wv >> (8 * k)) & 0xFF if k else wv & 0xFF
                        plsc.addupdate_scatter(cnt_v, [crows, bk], ones16)

                return carry

            lax.fori_loop(0, groups_per_chunk, group_body, 0)
            if chunk + 2 < n_chunks:
                cps.append(start_load(chunk + 2))

        pltpu.sync_copy(cnt_v, counts_hbm.at[pl.ds(base, bags_per_w), :])

    return counts_kernel(words)


def _tc_combine(counts, table_p, inv_l):
    B = counts.shape[0]
    D = table_p.shape[1]
    blk = 2048

    def mm(cnt_ref, tab_ref, o_ref):
        o_ref[...] = jnp.dot(
            cnt_ref[...], tab_ref[...],
            preferred_element_type=jnp.float32) * inv_l

    return pl.pallas_call(
        mm,
        grid=(B // blk,),
        in_specs=[
            pl.BlockSpec((blk, _VPAD), lambda i: (i, 0)),
            pl.BlockSpec((_VPAD, D), lambda i: (0, 0)),
        ],
        out_specs=pl.BlockSpec((blk, D), lambda i: (i, 0)),
        out_shape=jax.ShapeDtypeStruct((B, D), jnp.float32),
    )(counts, table_p)


def kernel(indices, table):
    B, L = indices.shape
    V, D = table.shape
    idx8 = indices.astype(jnp.uint8).reshape(B, L // 4, 4)
    words = jax.lax.bitcast_convert_type(idx8, jnp.int32)
    counts = _sc_counts(words, B, L // 4)
    table_p = jnp.zeros((_VPAD, D), table.dtype).at[:V, :].set(table)
    return _tc_combine(counts, table_p, 1.0 / L)


# speedup vs baseline: 1.2476x; 1.2476x over previous
import functools

import jax
import jax.numpy as jnp
from jax import lax
from jax.experimental import pallas as pl
from jax.experimental.pallas import tpu as pltpu
from jax.experimental.pallas import tpu_sc as plsc

_NC = 2
_NS = 16
_LANES = 16
_NW = _NC * _NS
_VPAD = 128


def _sc_counts(indices):
    Bh, L = indices.shape
    bags_per_w = Bh // _NW
    n_chunks = 2
    chunk_b = bags_per_w // n_chunks
    groups_per_chunk = chunk_b // _LANES

    mesh = plsc.VectorSubcoreMesh(
        core_axis_name="c", subcore_axis_name="s",
        num_cores=_NC, num_subcores=_NS)

    @functools.partial(
        pl.kernel,
        mesh=mesh,
        out_type=jax.ShapeDtypeStruct((Bh, _VPAD), jnp.float32),
        scratch_types=[
            pltpu.VMEM((chunk_b, L), jnp.int32),
            pltpu.VMEM((chunk_b, L), jnp.int32),
            pltpu.VMEM((bags_per_w, _VPAD), jnp.float32),
            pltpu.SemaphoreType.DMA,
            pltpu.SemaphoreType.DMA,
        ],
        compiler_params=pltpu.CompilerParams(
            needs_layout_passes=False, use_tc_tiling_on_sc=False),
    )
    def counts_kernel(idx_hbm, counts_hbm, idx_v0, idx_v1, cnt_v, sem0, sem1):
        wid = lax.axis_index("s") * _NC + lax.axis_index("c")
        base = wid * bags_per_w
        bufs = (idx_v0, idx_v1)
        sems = (sem0, sem1)

        def start_load(c):
            return pltpu.async_copy(
                idx_hbm.at[pl.ds(base + c * chunk_b, chunk_b), :],
                bufs[c % 2], sems[c % 2])

        cps = [start_load(0), start_load(1)]

        rows16 = lax.iota(jnp.int32, 16)
        ones16 = jnp.ones((16,), jnp.float32)
        zeros16 = jnp.zeros((16,), jnp.float32)

        @plsc.parallel_loop(0, bags_per_w, unroll=2)
        def _zero(r):
            for c in range(_VPAD // 16):
                cnt_v[r, pl.ds(c * 16, 16)] = zeros16

        for chunk in range(n_chunks):
            cps[chunk].wait()
            idx_v = bufs[chunk % 2]

            def group_body(g, carry, idx_v=idx_v, chunk=chunk):
                grows = g * _LANES + rows16
                crows = chunk * chunk_b + grows

                @plsc.parallel_loop(0, L, unroll=8)
                def _accum(l):
                    col = jnp.full((16,), l, jnp.int32)
                    idxv = plsc.load_gather(idx_v, [grows, col])
                    plsc.addupdate_scatter(cnt_v, [crows, idxv], ones16)

                return carry

            lax.fori_loop(0, groups_per_chunk, group_body, 0)
            if chunk + 2 < n_chunks:
                cps.append(start_load(chunk + 2))

        pltpu.sync_copy(cnt_v, counts_hbm.at[pl.ds(base, bags_per_w), :])

    return counts_kernel(indices)


def _tc_combine(counts, table_p, inv_l):
    Bh = counts.shape[0]
    D = table_p.shape[1]
    blk = 2048

    def mm(cnt_ref, tab_ref, o_ref):
        o_ref[...] = jnp.dot(
            cnt_ref[...], tab_ref[...],
            preferred_element_type=jnp.float32) * inv_l

    return pl.pallas_call(
        mm,
        grid=(Bh // blk,),
        in_specs=[
            pl.BlockSpec((blk, _VPAD), lambda i: (i, 0)),
            pl.BlockSpec((_VPAD, D), lambda i: (0, 0)),
        ],
        out_specs=pl.BlockSpec((blk, D), lambda i: (i, 0)),
        out_shape=jax.ShapeDtypeStruct((Bh, D), jnp.float32),
    )(counts, table_p)


def kernel(indices, table):
    B, L = indices.shape
    V, D = table.shape
    h = B // 2
    idx = indices.astype(jnp.int32)
    table_p = jnp.zeros((_VPAD, D), table.dtype).at[:V, :].set(table)
    outs = [_tc_combine(_sc_counts(idx[k * h:(k + 1) * h]), table_p, 1.0 / L)
            for k in range(2)]
    return jnp.concatenate(outs, axis=0)


# speedup vs baseline: 1.2534x; 1.0047x over previous
import functools

import jax
import jax.numpy as jnp
from jax import lax
from jax.experimental import pallas as pl
from jax.experimental.pallas import tpu as pltpu
from jax.experimental.pallas import tpu_sc as plsc

_NC = 2
_NS = 16
_LANES = 16
_NW = _NC * _NS
_VPAD = 128


def _sc_counts(indices):
    B, L = indices.shape
    bags_per_w = B // _NW
    n_chunks = 4
    chunk_b = bags_per_w // n_chunks
    groups_per_chunk = chunk_b // _LANES

    mesh = plsc.VectorSubcoreMesh(
        core_axis_name="c", subcore_axis_name="s",
        num_cores=_NC, num_subcores=_NS)

    @functools.partial(
        pl.kernel,
        mesh=mesh,
        out_type=jax.ShapeDtypeStruct((B, _VPAD), jnp.float32),
        scratch_types=[
            pltpu.VMEM((chunk_b, L), jnp.int32),
            pltpu.VMEM((chunk_b, L), jnp.int32),
            pltpu.VMEM((bags_per_w, _VPAD), jnp.float32),
            pltpu.SemaphoreType.DMA,
            pltpu.SemaphoreType.DMA,
        ],
        compiler_params=pltpu.CompilerParams(
            needs_layout_passes=False, use_tc_tiling_on_sc=False),
    )
    def counts_kernel(idx_hbm, counts_hbm, idx_v0, idx_v1, cnt_v, sem0, sem1):
        wid = lax.axis_index("s") * _NC + lax.axis_index("c")
        base = wid * bags_per_w
        bufs = (idx_v0, idx_v1)
        sems = (sem0, sem1)

        def start_load(c):
            return pltpu.async_copy(
                idx_hbm.at[pl.ds(base + c * chunk_b, chunk_b), :],
                bufs[c % 2], sems[c % 2])

        cps = [start_load(0), start_load(1)]

        rows16 = lax.iota(jnp.int32, 16)
        ones16 = jnp.ones((16,), jnp.float32)
        zeros16 = jnp.zeros((16,), jnp.float32)

        @plsc.parallel_loop(0, bags_per_w, unroll=2)
        def _zero(r):
            for c in range(_VPAD // 16):
                cnt_v[r, pl.ds(c * 16, 16)] = zeros16

        for chunk in range(n_chunks):
            cps[chunk].wait()
            idx_v = bufs[chunk % 2]

            def group_body(g, carry, idx_v=idx_v, chunk=chunk):
                grows = g * _LANES + rows16
                crows = chunk * chunk_b + grows

                @plsc.parallel_loop(0, L, unroll=16)
                def _accum(l):
                    col = jnp.full((16,), l, jnp.int32)
                    idxv = plsc.load_gather(idx_v, [grows, col])
                    plsc.addupdate_scatter(cnt_v, [crows, idxv], ones16)

                return carry

            lax.fori_loop(0, groups_per_chunk, group_body, 0)
            if chunk + 2 < n_chunks:
                cps.append(start_load(chunk + 2))

        pltpu.sync_copy(cnt_v, counts_hbm.at[pl.ds(base, bags_per_w), :])

    return counts_kernel(indices)


def _tc_combine(counts, table_p, inv_l):
    B = counts.shape[0]
    D = table_p.shape[1]
    blk = 1024

    def mm(cnt_ref, tab_ref, o_ref):
        o_ref[...] = jnp.dot(
            cnt_ref[...], tab_ref[...],
            preferred_element_type=jnp.float32) * inv_l

    return pl.pallas_call(
        mm,
        grid=(B // blk,),
        in_specs=[
            pl.BlockSpec((blk, _VPAD), lambda i: (i, 0)),
            pl.BlockSpec((_VPAD, D), lambda i: (0, 0)),
        ],
        out_specs=pl.BlockSpec((blk, D), lambda i: (i, 0)),
        out_shape=jax.ShapeDtypeStruct((B, D), jnp.float32),
    )(counts, table_p)


def kernel(indices, table):
    _, L = indices.shape
    V, D = table.shape
    counts = _sc_counts(indices.astype(jnp.int32))
    table_p = jnp.zeros((_VPAD, D), table.dtype).at[:V, :].set(table)
    return _tc_combine(counts, table_p, 1.0 / L)


# speedup vs baseline: 1.3752x; 1.0971x over previous
import functools

import jax
import jax.numpy as jnp
from jax import lax
from jax.experimental import pallas as pl
from jax.experimental.pallas import tpu as pltpu
from jax.experimental.pallas import tpu_sc as plsc

_NC = 2
_NS = 16
_LANES = 16
_NW = _NC * _NS
_VPAD = 128


def _sc_counts(indices):
    B, L = indices.shape
    bags_per_w = B // _NW
    n_chunks = 4
    chunk_b = bags_per_w // n_chunks
    groups_per_chunk = chunk_b // _LANES

    mesh = plsc.VectorSubcoreMesh(
        core_axis_name="c", subcore_axis_name="s",
        num_cores=_NC, num_subcores=_NS)

    @functools.partial(
        pl.kernel,
        mesh=mesh,
        out_type=jax.ShapeDtypeStruct((B, _VPAD), jnp.float32),
        scratch_types=[
            pltpu.VMEM((chunk_b, L), jnp.int32),
            pltpu.VMEM((chunk_b, L), jnp.int32),
            pltpu.VMEM((bags_per_w, _VPAD), jnp.float32),
            pltpu.SemaphoreType.DMA,
            pltpu.SemaphoreType.DMA,
        ],
        compiler_params=pltpu.CompilerParams(
            needs_layout_passes=False, use_tc_tiling_on_sc=False),
    )
    def counts_kernel(idx_hbm, counts_hbm, idx_v0, idx_v1, cnt_v, sem0, sem1):
        wid = lax.axis_index("s") * _NC + lax.axis_index("c")
        base = wid * bags_per_w
        bufs = (idx_v0, idx_v1)
        sems = (sem0, sem1)

        def start_load(c):
            return pltpu.async_copy(
                idx_hbm.at[pl.ds(base + c * chunk_b, chunk_b), :],
                bufs[c % 2], sems[c % 2])

        cps = [start_load(0), start_load(1)]

        rows16 = lax.iota(jnp.int32, 16)
        ones16 = jnp.ones((16,), jnp.float32)
        zeros16 = jnp.zeros((16,), jnp.float32)

        @plsc.parallel_loop(0, bags_per_w, unroll=2)
        def _zero(r):
            for c in range(_VPAD // 16):
                cnt_v[r, pl.ds(c * 16, 16)] = zeros16

        for chunk in range(n_chunks):
            cps[chunk].wait()
            idx_v = bufs[chunk % 2]

            def group_body(g, carry, idx_v=idx_v, chunk=chunk):
                grows = g * _LANES + rows16
                crows = chunk * chunk_b + grows

                @plsc.parallel_loop(0, L, unroll=8)
                def _accum(l):
                    col = jnp.full((16,), l, jnp.int32)
                    idxv = plsc.load_gather(idx_v, [grows, col])
                    plsc.addupdate_scatter(cnt_v, [crows, idxv], ones16)

                return carry

            lax.fori_loop(0, groups_per_chunk, group_body, 0)
            if chunk + 2 < n_chunks:
                cps.append(start_load(chunk + 2))

        pltpu.sync_copy(cnt_v, counts_hbm.at[pl.ds(base, bags_per_w), :])

    return counts_kernel(indices)


def _tc_combine(counts, table_p, inv_l):
    B = counts.shape[0]
    D = table_p.shape[1]
    blk = 2048

    def mm(cnt_ref, tab_ref, o_ref):
        o_ref[...] = jnp.dot(
            cnt_ref[...], tab_ref[...],
            preferred_element_type=jnp.float32) * inv_l

    return pl.pallas_call(
        mm,
        grid=(B // blk,),
        in_specs=[
            pl.BlockSpec((blk, _VPAD), lambda i: (i, 0)),
            pl.BlockSpec((_VPAD, D), lambda i: (0, 0)),
        ],
        out_specs=pl.BlockSpec((blk, D), lambda i: (i, 0)),
        out_shape=jax.ShapeDtypeStruct((B, D), jnp.float32),
    )(counts, table_p)


def kernel(indices, table):
    _, L = indices.shape
    V, D = table.shape
    counts = _sc_counts(indices.astype(jnp.int32))
    table_p = jnp.zeros((_VPAD, D), table.dtype).at[:V, :].set(table)
    return _tc_combine(counts, table_p, 1.0 / L)
